# baseline (speedup 1.0000x reference)
import jax
import jax.numpy as jnp
from jax.experimental import pallas as pl
from jax.experimental.pallas import tpu as pltpu

_N, _P, _B, _D, _M = 100000, 256, 4096, 128, 32
_SLOTS = 4
_CAP = 6400
_CHUNKS = [2400, 3200, 4800] + [6400] * 14
_STARTS = [sum(_CHUNKS[:j]) for j in range(len(_CHUNKS))]
assert sum(_CHUNKS) == _N and all(c % 8 == 0 for c in _CHUNKS)


def _body(meta_hbm, card_hbm, pools_hbm, w_ref, b_ref, out_ref,
          meta_buf, card_buf, pools_buf, acc_ref,
          meta_sems, card_sems, pool_sem):
    pltpu.make_async_copy(pools_hbm, pools_buf, pool_sem).start()

    def _copy(j, slot):
        st, sz = _STARTS[j], _CHUNKS[j]
        m = pltpu.make_async_copy(meta_hbm.at[pl.ds(st, sz), :],
                                  meta_buf.at[slot, pl.ds(0, sz), :],
                                  meta_sems.at[slot])
        c = pltpu.make_async_copy(card_hbm.at[pl.ds(st, sz), :],
                                  card_buf.at[slot, pl.ds(0, sz), :],
                                  card_sems.at[slot])
        return m, c

    def _start_prio(j, slot):
        st, sz = _STARTS[j], _CHUNKS[j]
        pltpu.async_copy(meta_hbm.at[pl.ds(st, sz), :],
                         meta_buf.at[slot, pl.ds(0, sz), :],
                         meta_sems.at[slot])
        pltpu.async_copy(card_hbm.at[pl.ds(st, sz), :],
                         card_buf.at[slot, pl.ds(0, sz), :],
                         card_sems.at[slot], priority=1)

    for k in range(_SLOTS):
        _start_prio(k, k)

    acc_ref[...] = jnp.zeros_like(acc_ref)

    for j, sz in enumerate(_CHUNKS):
        slot = j % _SLOTS
        m, c = _copy(j, slot)
        m.wait()
        c.wait()
        card_blk = card_buf[slot, pl.ds(0, sz), :]
        pre = jnp.dot(card_blk, w_ref[...],
                      preferred_element_type=jnp.float32) + b_ref[...]
        transformed = pre * jax.nn.sigmoid(pre)
        acc_ref[...] += jax.lax.dot_general(
            meta_buf[slot, pl.ds(0, sz), :].astype(jnp.bfloat16),
            transformed.astype(jnp.bfloat16),
            (((0,), (0,)), ((), ())),
            preferred_element_type=jnp.float32)
        if j + _SLOTS < len(_CHUNKS):
            _start_prio(j + _SLOTS, slot)

    pltpu.make_async_copy(pools_hbm, pools_buf, pool_sem).wait()
    out_ref[...] = jnp.dot(pools_buf[...], acc_ref[...],
                           preferred_element_type=jnp.float32)


def kernel(batch_pools, metapath, card_embeddings, W, b_dense):
    b2 = b_dense.reshape(1, _M)
    return pl.pallas_call(
        _body,
        in_specs=[
            pl.BlockSpec(memory_space=pl.ANY),
            pl.BlockSpec(memory_space=pl.ANY),
            pl.BlockSpec(memory_space=pl.ANY),
            pl.BlockSpec(memory_space=pltpu.VMEM),
            pl.BlockSpec(memory_space=pltpu.VMEM),
        ],
        out_specs=pl.BlockSpec(memory_space=pltpu.VMEM),
        out_shape=jax.ShapeDtypeStruct((_B, _M), jnp.float32),
        scratch_shapes=[
            pltpu.VMEM((_SLOTS, _CAP, _P), jnp.float32),
            pltpu.VMEM((_SLOTS, _CAP, _D), jnp.float32),
            pltpu.VMEM((_B, _P), jnp.float32),
            pltpu.VMEM((_P, _M), jnp.float32),
            pltpu.SemaphoreType.DMA((_SLOTS,)),
            pltpu.SemaphoreType.DMA((_SLOTS,)),
            pltpu.SemaphoreType.DMA,
        ],
    )(metapath, card_embeddings, batch_pools, W, b2)
